# baseline (speedup 1.0000x reference)
import functools

import jax
import jax.numpy as jnp
from jax import lax
from jax.experimental import pallas as pl
from jax.experimental.pallas import tpu as pltpu
from jax.experimental.pallas import tpu_sc as plsc

_R, _D = 1_000_000, 64
_N = 16384
_FLAT = _N * _D

_NC, _NS, _L = 2, 16, 16
_NW = _NC * _NS

_CH = 128
_ROWS = _FLAT // _CH
_NR = _ROWS // _NW


def _gather_body(t_hbm, c_hbm, out_hbm, ibuf, gbuf, sem):
    wid = lax.axis_index("s") * _NC + lax.axis_index("c")
    row0 = wid * _NR
    pltpu.sync_copy(c_hbm.at[pl.ds(row0, _NR), :], ibuf)

    def fire(r, carry):
        jconst = ((row0 + r) >> 7) * _R
        for m in range(_CH // _L):
            sl = pl.ds(m * _L, _L)
            ibuf[r, sl] = ibuf[r, sl] + jconst
        pltpu.async_copy(t_hbm.at[ibuf.at[r]], gbuf.at[r], sem)
        return carry

    lax.fori_loop(0, _NR, fire, 0)
    pltpu.make_async_copy(out_hbm.at[pl.ds(row0, _NR), :], gbuf, sem).wait()
    pltpu.sync_copy(gbuf, out_hbm.at[pl.ds(row0, _NR), :])


@functools.cache
def _gather_kernel():
    mesh = plsc.VectorSubcoreMesh(
        core_axis_name="c", subcore_axis_name="s", num_cores=_NC, num_subcores=_NS
    )
    return pl.kernel(
        _gather_body,
        mesh=mesh,
        out_type=jax.ShapeDtypeStruct((_ROWS, _CH), jnp.float32),
        scratch_types=[
            pltpu.VMEM((_NR, _CH), jnp.int32),
            pltpu.VMEM((_NR, _CH), jnp.float32),
            pltpu.SemaphoreType.DMA,
        ],
    )


def kernel(t, d, c):
    idx = c + jnp.asarray(d, dtype=c.dtype)
    cflat = idx.T.reshape(_ROWS, _CH)
    out = _gather_kernel()(t.T.reshape(_R * _D), cflat)
    return out.reshape(_D, _N).T


# speedup vs baseline: 10.1984x; 10.1984x over previous
import functools

import jax
import jax.numpy as jnp
from jax import lax
from jax.experimental import pallas as pl
from jax.experimental.pallas import tpu as pltpu
from jax.experimental.pallas import tpu_sc as plsc

_R, _D = 1_000_000, 64
_N = 16384
_FLAT = _N * _D

_NC, _NS, _L = 2, 16, 16
_NW = _NC * _NS

_CH = 128
_ROWS = _FLAT // _CH
_NR = _ROWS // _NW

_K = 13
_BW = _K * _CH
_NB = 7813 // _K
_OB = _D * _BW


def _relayout_body(in_ref, out_ref):
    x = in_ref[...].reshape(_D, _K, _CH)
    out_ref[...] = jnp.swapaxes(x, 0, 1).reshape(_OB)


@functools.cache
def _relayout_kernel():
    return pl.pallas_call(
        _relayout_body,
        grid=(_NB,),
        in_specs=[pl.BlockSpec((_D, _BW), lambda c: (0, c))],
        out_specs=pl.BlockSpec((_OB,), lambda c: (c,)),
        out_shape=jax.ShapeDtypeStruct((_NB * _OB,), jnp.float32),
    )


def _gather_body(t_hbm, c_hbm, out_hbm, ibuf, gbuf, sem):
    wid = lax.axis_index("s") * _NC + lax.axis_index("c")
    row0 = wid * _NR
    pltpu.sync_copy(c_hbm.at[pl.ds(row0, _NR), :], ibuf)

    def fire(r, carry):
        jconst = ((row0 + r) >> 7) * _CH
        for m in range(_CH // _L):
            sl = pl.ds(m * _L, _L)
            v = ibuf[r, sl]
            ibuf[r, sl] = ((v & ~jnp.int32(127)) << 6) + ((v & 127) + jconst)
        pltpu.async_copy(t_hbm.at[ibuf.at[r]], gbuf.at[r], sem)
        return carry

    lax.fori_loop(0, _NR, fire, 0)
    pltpu.make_async_copy(out_hbm.at[pl.ds(row0, _NR), :], gbuf, sem).wait()
    pltpu.sync_copy(gbuf, out_hbm.at[pl.ds(row0, _NR), :])


@functools.cache
def _gather_kernel():
    mesh = plsc.VectorSubcoreMesh(
        core_axis_name="c", subcore_axis_name="s", num_cores=_NC, num_subcores=_NS
    )
    return pl.kernel(
        _gather_body,
        mesh=mesh,
        out_type=jax.ShapeDtypeStruct((_ROWS, _CH), jnp.float32),
        scratch_types=[
            pltpu.VMEM((_NR, _CH), jnp.int32),
            pltpu.VMEM((_NR, _CH), jnp.float32),
            pltpu.SemaphoreType.DMA,
        ],
    )


def kernel(t, d, c):
    idx = c + jnp.asarray(d, dtype=c.dtype)
    cflat = idx.T.reshape(_ROWS, _CH)
    tflat = _relayout_kernel()(t.T)
    out = _gather_kernel()(tflat, cflat)
    return out.reshape(_D, _N).T


# speedup vs baseline: 21.3793x; 2.0963x over previous
import functools

import jax
import jax.numpy as jnp
from jax import lax
from jax.experimental import pallas as pl
from jax.experimental.pallas import tpu as pltpu
from jax.experimental.pallas import tpu_sc as plsc

_R, _D = 1_000_000, 64
_N = 16384
_FLAT = _N * _D

_NC, _NS, _L = 2, 16, 16
_NW = _NC * _NS

_CH = 128
_ROWS = _FLAT // _CH
_NR = _ROWS // _NW

_K = 104
_BW = _K * _CH
_NB = -(-7813 // _K)
_OB = _D * _BW


def _relayout_body(in_ref, out_ref):
    x = in_ref[...].reshape(_D, _K, _CH)
    out_ref[...] = jnp.swapaxes(x, 0, 1).reshape(_OB)


@functools.cache
def _relayout_kernel():
    return pl.pallas_call(
        _relayout_body,
        grid=(_NB,),
        in_specs=[pl.BlockSpec((_D, _BW), lambda c: (0, c))],
        out_specs=pl.BlockSpec((_OB,), lambda c: (c,)),
        out_shape=jax.ShapeDtypeStruct((_NB * _OB,), jnp.float32),
    )


def _gather_body(t_hbm, c_hbm, out_hbm, ibuf, gbuf, sem):
    wid = lax.axis_index("s") * _NC + lax.axis_index("c")
    row0 = wid * _NR
    pltpu.sync_copy(c_hbm.at[pl.ds(row0, _NR), :], ibuf)

    def fire(r, carry):
        jconst = ((row0 + r) >> 7) * _CH
        for m in range(_CH // _L):
            sl = pl.ds(m * _L, _L)
            v = ibuf[r, sl]
            ibuf[r, sl] = ((v & ~jnp.int32(127)) << 6) + ((v & 127) + jconst)
        pltpu.async_copy(t_hbm.at[ibuf.at[r]], gbuf.at[r], sem)
        return carry

    lax.fori_loop(0, _NR, fire, 0)
    pltpu.make_async_copy(out_hbm.at[pl.ds(row0, _NR), :], gbuf, sem).wait()
    pltpu.sync_copy(gbuf, out_hbm.at[pl.ds(row0, _NR), :])


@functools.cache
def _gather_kernel():
    mesh = plsc.VectorSubcoreMesh(
        core_axis_name="c", subcore_axis_name="s", num_cores=_NC, num_subcores=_NS
    )
    return pl.kernel(
        _gather_body,
        mesh=mesh,
        out_type=jax.ShapeDtypeStruct((_ROWS, _CH), jnp.float32),
        scratch_types=[
            pltpu.VMEM((_NR, _CH), jnp.int32),
            pltpu.VMEM((_NR, _CH), jnp.float32),
            pltpu.SemaphoreType.DMA,
        ],
    )


def kernel(t, d, c):
    idx = c + jnp.asarray(d, dtype=c.dtype)
    cflat = idx.T.reshape(_ROWS, _CH)
    tflat = _relayout_kernel()(t.T)
    out = _gather_kernel()(tflat, cflat)
    return out.reshape(_D, _N).T


# speedup vs baseline: 22.1254x; 1.0349x over previous
import functools

import jax
import jax.numpy as jnp
from jax import lax
from jax.experimental import pallas as pl
from jax.experimental.pallas import tpu as pltpu
from jax.experimental.pallas import tpu_sc as plsc

_R, _D = 1_000_000, 64
_N = 16384
_FLAT = _N * _D

_NC, _NS, _L = 2, 16, 16
_NW = _NC * _NS

_CH = 128
_ROWS = _FLAT // _CH
_NR = _ROWS // _NW

_K = 256
_BW = _K * _CH
_NB = -(-7813 // _K)
_OB = _D * _BW


def _relayout_body(in_ref, out_ref):
    x = in_ref[...].reshape(_D, _K, _CH)
    out_ref[...] = jnp.swapaxes(x, 0, 1).reshape(_OB)


@functools.cache
def _relayout_kernel():
    return pl.pallas_call(
        _relayout_body,
        grid=(_NB,),
        in_specs=[pl.BlockSpec((_D, _BW), lambda c: (0, c))],
        out_specs=pl.BlockSpec((_OB,), lambda c: (c,)),
        out_shape=jax.ShapeDtypeStruct((_NB * _OB,), jnp.float32),
    )


def _gather_body(t_hbm, c_hbm, out_hbm, ibuf, gbuf, sem):
    wid = lax.axis_index("s") * _NC + lax.axis_index("c")
    row0 = wid * _NR
    pltpu.sync_copy(c_hbm.at[pl.ds(row0, _NR), :], ibuf)

    def fire(r, carry):
        jconst = ((row0 + r) >> 7) * _CH
        for m in range(_CH // _L):
            sl = pl.ds(m * _L, _L)
            v = ibuf[r, sl]
            ibuf[r, sl] = ((v & ~jnp.int32(127)) << 6) + ((v & 127) + jconst)
        pltpu.async_copy(t_hbm.at[ibuf.at[r]], gbuf.at[r], sem)
        return carry

    lax.fori_loop(0, _NR, fire, 0)
    pltpu.make_async_copy(out_hbm.at[pl.ds(row0, _NR), :], gbuf, sem).wait()
    pltpu.sync_copy(gbuf, out_hbm.at[pl.ds(row0, _NR), :])


@functools.cache
def _gather_kernel():
    mesh = plsc.VectorSubcoreMesh(
        core_axis_name="c", subcore_axis_name="s", num_cores=_NC, num_subcores=_NS
    )
    return pl.kernel(
        _gather_body,
        mesh=mesh,
        out_type=jax.ShapeDtypeStruct((_ROWS, _CH), jnp.float32),
        scratch_types=[
            pltpu.VMEM((_NR, _CH), jnp.int32),
            pltpu.VMEM((_NR, _CH), jnp.float32),
            pltpu.SemaphoreType.DMA,
        ],
    )


def kernel(t, d, c):
    idx = c + jnp.asarray(d, dtype=c.dtype)
    cflat = idx.T.reshape(_ROWS, _CH)
    tflat = _relayout_kernel()(t.T)
    out = _gather_kernel()(tflat, cflat)
    return out.reshape(_D, _N).T
